# baseline (speedup 1.0000x reference)
import jax
import jax.numpy as jnp
from jax import lax
from jax.experimental import pallas as pl
from jax.experimental.pallas import tpu as pltpu
from jax.experimental.pallas import tpu_sc as plsc

D = 64
B = 16384
K = 20

NC, NS, L = 2, 16, 16
NW = NC * NS
PER_W = B // NW
CHUNK = 32
NSTEP = PER_W // CHUNK
NNEG = CHUNK * K
NGATH = NNEG // 128
NV = D // L


def _sc_scores_body(cen_hbm, ctx_hbm, neg_hbm, win_hbm, wout_hbm,
                    pos_out, neg_out,
                    cidx, xidx, nidx, cp_i, xp_i, np_i,
                    crows, prows, nrows, psc, nsc, sem):
    wid = lax.axis_index("s") * NC + lax.axis_index("c")

    def step(c, _):
        base = wid * PER_W + c * CHUNK
        cps = [pltpu.async_copy(cen_hbm.at[pl.ds(base, CHUNK)], cidx, sem),
               pltpu.async_copy(ctx_hbm.at[pl.ds(base, CHUNK)], xidx, sem)]
        for k in range(K):
            cps.append(pltpu.async_copy(neg_hbm.at[pl.ds(k * B + base, CHUNK)],
                                        nidx.at[pl.ds(k * CHUNK, CHUNK)], sem))
        for cp in cps:
            cp.wait()
        for j in range(CHUNK // L):
            cp_i[pl.ds(j * L, L)] = lax.shift_right_logical(
                cidx[pl.ds(j * L, L)], 1)
            xp_i[pl.ds(j * L, L)] = lax.shift_right_logical(
                xidx[pl.ds(j * L, L)], 1)
        for j in range(NNEG // L):
            np_i[pl.ds(j * L, L)] = lax.shift_right_logical(
                nidx[pl.ds(j * L, L)], 1)
        cps = [pltpu.async_copy(win_hbm.at[cp_i], crows, sem),
               pltpu.async_copy(wout_hbm.at[xp_i], prows, sem)]
        for j in range(NGATH):
            cps.append(pltpu.async_copy(wout_hbm.at[np_i.at[pl.ds(j * 128, 128)]],
                                        nrows.at[pl.ds(j * 128, 128)], sem))
        for cp in cps:
            cp.wait()

        lane15 = lax.iota(jnp.int32, L) == (L - 1)

        def halves(row_vecs, lo, hi):
            slo = row_vecs[0] * lo[0]
            shi = row_vecs[4] * hi[0]
            for j in range(1, NV):
                slo = slo + row_vecs[j] * lo[j]
                shi = shi + row_vecs[j + 4] * hi[j]
            return slo, shi

        def brow(b, _):
            hc = (plsc.load_gather(cidx, [jnp.full((L,), b, jnp.int32)])
                  & 1).astype(jnp.float32)
            hx = (plsc.load_gather(xidx, [jnp.full((L,), b, jnp.int32)])
                  & 1).astype(jnp.float32)
            cv = [crows[b, pl.ds(j * L, L)] for j in range(2 * NV)]
            ce = [cv[j] + (cv[j + 4] - cv[j]) * hc for j in range(NV)]
            pv = [prows[b, pl.ds(j * L, L)] for j in range(2 * NV)]
            slo, shi = halves(pv, ce, ce)
            s = plsc.cumsum(slo) * (1.0 - hx) + plsc.cumsum(shi) * hx
            plsc.store_scatter(psc, [jnp.full((L,), c * CHUNK + b, jnp.int32)],
                               s, mask=lane15)
            for k in range(K):
                hn = (plsc.load_gather(
                    nidx, [jnp.full((L,), k * CHUNK + b, jnp.int32)])
                    & 1).astype(jnp.float32)
                nv = [nrows[k * CHUNK + b, pl.ds(j * L, L)]
                      for j in range(2 * NV)]
                tlo, thi = halves(nv, ce, ce)
                t = plsc.cumsum(tlo) * (1.0 - hn) + plsc.cumsum(thi) * hn
                plsc.store_scatter(
                    nsc, [jnp.full((L,), k * PER_W + c * CHUNK + b, jnp.int32)],
                    t, mask=lane15)
            return 0

        lax.fori_loop(0, CHUNK, brow, 0)
        return 0

    lax.fori_loop(0, NSTEP, step, 0)
    pltpu.sync_copy(psc, pos_out.at[pl.ds(wid * PER_W, PER_W)])
    for k in range(K):
        pltpu.sync_copy(nsc.at[pl.ds(k * PER_W, PER_W)],
                        neg_out.at[pl.ds(k * B + wid * PER_W, PER_W)])


@jax.jit
def _sc_scores(cen, ctx, neg1d, w_in2, w_out2):
    f = pl.kernel(
        _sc_scores_body,
        out_type=(jax.ShapeDtypeStruct((B,), jnp.float32),
                  jax.ShapeDtypeStruct((B * K,), jnp.float32)),
        mesh=plsc.VectorSubcoreMesh(core_axis_name="c", subcore_axis_name="s"),
        compiler_params=pltpu.CompilerParams(needs_layout_passes=False),
        scratch_types=[
            pltpu.VMEM((CHUNK,), jnp.int32),
            pltpu.VMEM((CHUNK,), jnp.int32),
            pltpu.VMEM((NNEG,), jnp.int32),
            pltpu.VMEM((CHUNK,), jnp.int32),
            pltpu.VMEM((CHUNK,), jnp.int32),
            pltpu.VMEM((NNEG,), jnp.int32),
            pltpu.VMEM((CHUNK, 2 * D), jnp.float32),
            pltpu.VMEM((CHUNK, 2 * D), jnp.float32),
            pltpu.VMEM((NNEG, 2 * D), jnp.float32),
            pltpu.VMEM((PER_W,), jnp.float32),
            pltpu.VMEM((K * PER_W,), jnp.float32),
            pltpu.SemaphoreType.DMA,
        ],
    )
    return f(cen, ctx, neg1d, w_in2, w_out2)


def _tc_loss_body(pos_ref, neg_ref, out_ref):
    pls = jax.nn.log_sigmoid(pos_ref[...])
    nls = jax.nn.log_sigmoid(-neg_ref[...])
    out_ref[0, 0] = -(jnp.sum(pls) + jnp.sum(nls)) / B


def _tc_loss(pos2d, neg2d):
    return pl.pallas_call(
        _tc_loss_body,
        out_shape=jax.ShapeDtypeStruct((1, 1), jnp.float32),
        out_specs=pl.BlockSpec(memory_space=pltpu.SMEM),
    )(pos2d, neg2d)


def kernel(center, context, negatives, W_in, W_out):
    cen = center.astype(jnp.int32)
    ctx = context.astype(jnp.int32)
    neg = negatives.astype(jnp.int32).T.reshape(B * K)
    w_in2 = W_in.reshape(W_in.shape[0] // 2, 2 * D)
    w_out2 = W_out.reshape(W_out.shape[0] // 2, 2 * D)
    pos_s, neg_s = _sc_scores(cen, ctx, neg, w_in2, w_out2)
    loss = _tc_loss(pos_s.reshape(B // 128, 128),
                    neg_s.reshape(B * K // 128, 128))
    return loss[0, 0]


# speedup vs baseline: 1.0795x; 1.0795x over previous
import jax
import jax.numpy as jnp
from jax import lax
from jax.experimental import pallas as pl
from jax.experimental.pallas import tpu as pltpu
from jax.experimental.pallas import tpu_sc as plsc

D = 64
B = 16384
K = 20

NC, NS, L = 2, 16, 16
NW = NC * NS
PER_W = B // NW
CHUNK = 32
NSTEP = PER_W // CHUNK
NNEG = CHUNK * K
NGATH = NNEG // 128


def _sc_scores_body(cen_hbm, ctx_hbm, neg_hbm, win_hbm, wout_hbm,
                    pos_out, neg_out,
                    cidx, xidx, nidx, crows, prows, nrows, psc, nsc,
                    isem, gsem):
    wid = lax.axis_index("s") * NC + lax.axis_index("c")
    lane15 = lax.iota(jnp.int32, L) == (L - 1)

    def stage_idx(c, p):
        base = wid * PER_W + c * CHUNK
        cps = [pltpu.async_copy(cen_hbm.at[pl.ds(base, CHUNK)], cidx[p], isem),
               pltpu.async_copy(ctx_hbm.at[pl.ds(base, CHUNK)], xidx[p], isem)]
        for k in range(K):
            cps.append(pltpu.async_copy(
                neg_hbm.at[pl.ds(k * B + base, CHUNK)],
                nidx[p].at[pl.ds(k * CHUNK, CHUNK)], isem))
        return cps

    def gather_rows(p):
        cps = [pltpu.async_copy(win_hbm.at[cidx[p]], crows[p], gsem),
               pltpu.async_copy(wout_hbm.at[xidx[p]], prows[p], gsem)]
        for j in range(NGATH):
            cps.append(pltpu.async_copy(
                wout_hbm.at[nidx[p].at[pl.ds(j * 128, 128)]],
                nrows[p].at[pl.ds(j * 128, 128)], gsem))
        return cps

    def compute(c, p):
        def brow(b, _):
            cvs = [crows[p][b, pl.ds(j * L, L)] for j in range(D // L)]
            pvs = [prows[p][b, pl.ds(j * L, L)] for j in range(D // L)]
            s = plsc.cumsum(sum(cv * pv for cv, pv in zip(cvs, pvs)))
            plsc.store_scatter(psc, [jnp.full((L,), c * CHUNK + b, jnp.int32)],
                               s, mask=lane15)
            for k in range(K):
                nvs = [nrows[p][k * CHUNK + b, pl.ds(j * L, L)]
                       for j in range(D // L)]
                t = plsc.cumsum(sum(cv * nv for cv, nv in zip(cvs, nvs)))
                plsc.store_scatter(
                    nsc, [jnp.full((L,), k * PER_W + c * CHUNK + b, jnp.int32)],
                    t, mask=lane15)
            return 0

        lax.fori_loop(0, CHUNK, brow, 0)

    idx_cps = stage_idx(0, 0)
    gat_cps = None
    for c in range(NSTEP):
        p = c % 2
        for cp in idx_cps:
            cp.wait()
        if gat_cps:
            for cp in gat_cps:
                cp.wait()
        idx_cps = stage_idx(c + 1, 1 - p) if c + 1 < NSTEP else []
        gat_cps = gather_rows(p)
        if c > 0:
            compute(c - 1, 1 - p)
    for cp in gat_cps:
        cp.wait()
    compute(NSTEP - 1, (NSTEP - 1) % 2)

    pltpu.sync_copy(psc, pos_out.at[pl.ds(wid * PER_W, PER_W)])
    for k in range(K):
        pltpu.sync_copy(nsc.at[pl.ds(k * PER_W, PER_W)],
                        neg_out.at[pl.ds(k * B + wid * PER_W, PER_W)])


@jax.jit
def _sc_scores(cen, ctx, neg1d, w_in, w_out):
    f = pl.kernel(
        _sc_scores_body,
        out_type=(jax.ShapeDtypeStruct((B,), jnp.float32),
                  jax.ShapeDtypeStruct((B * K,), jnp.float32)),
        mesh=plsc.VectorSubcoreMesh(core_axis_name="c", subcore_axis_name="s"),
        compiler_params=pltpu.CompilerParams(needs_layout_passes=False,
                                             use_tc_tiling_on_sc=False),
        scratch_types=[
            [pltpu.VMEM((CHUNK,), jnp.int32)] * 2,
            [pltpu.VMEM((CHUNK,), jnp.int32)] * 2,
            [pltpu.VMEM((NNEG,), jnp.int32)] * 2,
            [pltpu.VMEM((CHUNK, D), jnp.float32)] * 2,
            [pltpu.VMEM((CHUNK, D), jnp.float32)] * 2,
            [pltpu.VMEM((NNEG, D), jnp.float32)] * 2,
            pltpu.VMEM((PER_W,), jnp.float32),
            pltpu.VMEM((K * PER_W,), jnp.float32),
            pltpu.SemaphoreType.DMA,
            pltpu.SemaphoreType.DMA,
        ],
    )
    return f(cen, ctx, neg1d, w_in, w_out)


def _tc_loss_body(pos_ref, neg_ref, out_ref):
    pls = jax.nn.log_sigmoid(pos_ref[...])
    nls = jax.nn.log_sigmoid(-neg_ref[...])
    out_ref[0, 0] = -(jnp.sum(pls) + jnp.sum(nls)) / B


def _tc_loss(pos2d, neg2d):
    return pl.pallas_call(
        _tc_loss_body,
        out_shape=jax.ShapeDtypeStruct((1, 1), jnp.float32),
        out_specs=pl.BlockSpec(memory_space=pltpu.SMEM),
    )(pos2d, neg2d)


def kernel(center, context, negatives, W_in, W_out):
    cen = center.astype(jnp.int32)
    ctx = context.astype(jnp.int32)
    neg = negatives.astype(jnp.int32).T.reshape(B * K)
    pos_s, neg_s = _sc_scores(cen, ctx, neg, W_in, W_out)
    loss = _tc_loss(pos_s.reshape(B // 128, 128),
                    neg_s.reshape(B * K // 128, 128))
    return loss[0, 0]


# speedup vs baseline: 1.2932x; 1.1979x over previous
import jax
import jax.numpy as jnp
from jax import lax
from jax.experimental import pallas as pl
from jax.experimental.pallas import tpu as pltpu
from jax.experimental.pallas import tpu_sc as plsc

VOCAB = 1000000
D = 64
B = 16384
K = 20

NC, NS, L = 2, 16, 16
NW = NC * NS
PER_W = B // NW
CHUNK = 32
NSTEP = PER_W // CHUNK
NNEG = CHUNK * K
NGATH = NNEG // 128
BCH = 128


def _sc_center_body(cen_hbm, win_hbm, cen_out, cidx, blk, ctile, sem):
    wid = lax.axis_index("s") * NC + lax.axis_index("c")
    tbase = wid * PER_W
    pltpu.sync_copy(cen_hbm.at[pl.ds(tbase, PER_W)], cidx)
    lanes = lax.iota(jnp.int32, L)

    def grp(g, _):
        vec = cidx[pl.ds(g * L, L)]
        rs, cps = [], []
        for i in range(L):
            r = jnp.max(jnp.where(lanes == i, vec, 0))
            rs.append(r)
            r0 = pl.multiple_of(lax.shift_right_logical(r, 3) * 8, 8)
            cps.append(pltpu.async_copy(win_hbm.at[pl.ds(r0, 8), :],
                                        blk.at[i], sem))
        for cp in cps:
            cp.wait()
        for i in range(L):
            row = rs[i] & 7
            for j in range(D // L):
                ctile[g * L + i, pl.ds(j * L, L)] = blk[i, row, pl.ds(j * L, L)]
        return 0

    lax.fori_loop(0, PER_W // L, grp, 0)
    pltpu.sync_copy(ctile, cen_out.at[pl.ds(tbase, PER_W)])


@jax.jit
def _sc_center(cen, w_in):
    f = pl.kernel(
        _sc_center_body,
        out_type=jax.ShapeDtypeStruct((B, D), jnp.float32),
        mesh=plsc.VectorSubcoreMesh(core_axis_name="c", subcore_axis_name="s"),
        compiler_params=pltpu.CompilerParams(needs_layout_passes=False),
        scratch_types=[
            pltpu.VMEM((PER_W,), jnp.int32),
            pltpu.VMEM((L, 8, D), jnp.float32),
            pltpu.VMEM((PER_W, D), jnp.float32),
            pltpu.SemaphoreType.DMA,
        ],
    )
    return f(cen, w_in)


def _sc_scores_body(cen_hbm, ctx_hbm, neg_hbm, wout_hbm,
                    pos_out, neg_out,
                    xidx, nidx, crows, prows, nrows, psc, nsc,
                    isem, gsem):
    wid = lax.axis_index("s") * NC + lax.axis_index("c")
    lane15 = lax.iota(jnp.int32, L) == (L - 1)

    def stage_idx(c, p):
        base = wid * PER_W + c * CHUNK
        cps = [pltpu.async_copy(ctx_hbm.at[pl.ds(base, CHUNK)], xidx[p], isem),
               pltpu.async_copy(cen_hbm.at[pl.ds(base, CHUNK)], crows[p], isem)]
        for k in range(K):
            cps.append(pltpu.async_copy(
                neg_hbm.at[pl.ds(k * B + base, CHUNK)],
                nidx[p].at[pl.ds(k * CHUNK, CHUNK)], isem))
        return cps

    def gather_rows(p):
        cps = [pltpu.async_copy(wout_hbm.at[xidx[p]], prows[p], gsem)]
        for j in range(NGATH):
            cps.append(pltpu.async_copy(
                wout_hbm.at[nidx[p].at[pl.ds(j * 128, 128)]],
                nrows[p].at[pl.ds(j * 128, 128)], gsem))
        return cps

    def compute(c, p):
        def brow(b, _):
            cvs = [crows[p][b, pl.ds(j * L, L)] for j in range(D // L)]
            pvs = [prows[p][b, pl.ds(j * L, L)] for j in range(D // L)]
            s = plsc.cumsum(sum(cv * pv for cv, pv in zip(cvs, pvs)))
            plsc.store_scatter(psc, [jnp.full((L,), c * CHUNK + b, jnp.int32)],
                               s, mask=lane15)
            for k in range(K):
                nvs = [nrows[p][k * CHUNK + b, pl.ds(j * L, L)]
                       for j in range(D // L)]
                t = plsc.cumsum(sum(cv * nv for cv, nv in zip(cvs, nvs)))
                plsc.store_scatter(
                    nsc, [jnp.full((L,), k * PER_W + c * CHUNK + b, jnp.int32)],
                    t, mask=lane15)
            return 0

        lax.fori_loop(0, CHUNK, brow, 0)

    idx_cps = stage_idx(0, 0)
    gat_cps = None
    for c in range(NSTEP):
        p = c % 2
        for cp in idx_cps:
            cp.wait()
        if gat_cps:
            for cp in gat_cps:
                cp.wait()
        idx_cps = stage_idx(c + 1, 1 - p) if c + 1 < NSTEP else []
        gat_cps = gather_rows(p)
        if c > 0:
            compute(c - 1, 1 - p)
    for cp in gat_cps:
        cp.wait()
    compute(NSTEP - 1, (NSTEP - 1) % 2)

    pltpu.sync_copy(psc, pos_out.at[pl.ds(wid * PER_W, PER_W)])
    for k in range(K):
        pltpu.sync_copy(nsc.at[pl.ds(k * PER_W, PER_W)],
                        neg_out.at[pl.ds(k * B + wid * PER_W, PER_W)])


@jax.jit
def _sc_scores(cen_emb, ctx, neg1d, w_out):
    f = pl.kernel(
        _sc_scores_body,
        out_type=(jax.ShapeDtypeStruct((B,), jnp.float32),
                  jax.ShapeDtypeStruct((B * K,), jnp.float32)),
        mesh=plsc.VectorSubcoreMesh(core_axis_name="c", subcore_axis_name="s"),
        compiler_params=pltpu.CompilerParams(needs_layout_passes=False,
                                             use_tc_tiling_on_sc=False),
        scratch_types=[
            [pltpu.VMEM((CHUNK,), jnp.int32)] * 2,
            [pltpu.VMEM((NNEG,), jnp.int32)] * 2,
            [pltpu.VMEM((CHUNK, D), jnp.float32)] * 2,
            [pltpu.VMEM((CHUNK, D), jnp.float32)] * 2,
            [pltpu.VMEM((NNEG, D), jnp.float32)] * 2,
            pltpu.VMEM((PER_W,), jnp.float32),
            pltpu.VMEM((K * PER_W,), jnp.float32),
            pltpu.SemaphoreType.DMA,
            pltpu.SemaphoreType.DMA,
        ],
    )
    return f(cen_emb, ctx, neg1d, w_out)


def _tc_loss_body(pos_ref, neg_ref, out_ref):
    pls = jax.nn.log_sigmoid(pos_ref[...])
    nls = jax.nn.log_sigmoid(-neg_ref[...])
    out_ref[0, 0] = -(jnp.sum(pls) + jnp.sum(nls)) / B


def _tc_loss(pos2d, neg2d):
    return pl.pallas_call(
        _tc_loss_body,
        out_shape=jax.ShapeDtypeStruct((1, 1), jnp.float32),
        out_specs=pl.BlockSpec(memory_space=pltpu.SMEM),
    )(pos2d, neg2d)


def kernel(center, context, negatives, W_in, W_out):
    cen = center.astype(jnp.int32)
    ctx = context.astype(jnp.int32)
    neg = negatives.astype(jnp.int32).T.reshape(B * K)
    cen_emb = _sc_center(cen, W_in)
    pos_s, neg_s = _sc_scores(cen_emb, ctx, neg, W_out)
    loss = _tc_loss(pos_s.reshape(B // 128, 128),
                    neg_s.reshape(B * K // 128, 128))
    return loss[0, 0]


# speedup vs baseline: 1.5597x; 1.2061x over previous
import jax
import jax.numpy as jnp
from jax import lax
from jax.experimental import pallas as pl
from jax.experimental.pallas import tpu as pltpu
from jax.experimental.pallas import tpu_sc as plsc

VOCAB = 1000000
D = 64
B = 16384
K = 20

NC, NS, L = 2, 16, 16
NW = NC * NS
PER_W = B // NW
CHUNK = 32
NSTEP = PER_W // CHUNK
NNEG = CHUNK * K
NGATH = NNEG // 128
BCH = 128


GB = 4
CLAST = (VOCAB - 128) // 128 * 128
TAIL0 = CLAST + 128


def _sc_center_body(cen_hbm, wt_hbm, tail_hbm, cen_out,
                    cidx, blk, ctile, wtail, sems):
    wid = lax.axis_index("s") * NC + lax.axis_index("c")
    tbase = wid * PER_W
    pltpu.sync_copy(cen_hbm.at[pl.ds(tbase, PER_W)], cidx.at[pl.ds(0, PER_W)])
    pltpu.sync_copy(tail_hbm, wtail)
    lanes = lax.iota(jnp.int32, L)

    def rof(b):
        vec = cidx[pl.ds(b & ~(L - 1), L)]
        return jnp.max(jnp.where(lanes == (b & (L - 1)), vec, 0))

    def c0of(r):
        return jnp.minimum(lax.shift_right_logical(r, 7), CLAST // 128) * 128

    def tiles(b, sg):
        c0 = pl.multiple_of(c0of(rof(b)), 128)
        return [(wt_hbm.at[pl.ds(t * 8, 8), pl.ds(c0, 128)],
                 blk.at[pl.ds(sg * D + t * 8, 8)]) for t in range(D // 8)]

    def issue(b, sg):
        for src, dst in tiles(b, sg):
            pltpu.async_copy(src, dst, sems[sg])

    def drain(b, sg):
        for src, dst in tiles(b, sg):
            pltpu.make_async_copy(src, dst, sems[sg]).wait()

    def extract(b, sg):
        r = rof(b)
        col = jnp.minimum(r - c0of(r), 127)
        rt = jnp.minimum(jnp.maximum(r - TAIL0, 0), 63)
        is_tail = r >= TAIL0
        csp = jnp.full((L,), col, jnp.int32)
        for j in range(D // L):
            v = plsc.load_gather(blk, [sg * D + j * L + lanes, csp])
            tv = wtail[rt, pl.ds(j * L, L)]
            ctile[b, pl.ds(j * L, L)] = jnp.where(is_tail, tv, v)

    for sg in range(GB):
        issue(sg, sg)

    def step(g, _):
        for sg in range(GB):
            bp = (g - 1) * GB + sg
            drain(bp, sg)
            extract(bp, sg)
            issue(g * GB + sg, sg)
        return 0

    lax.fori_loop(1, PER_W // GB, step, 0)
    for sg in range(GB):
        bp = PER_W - GB + sg
        drain(bp, sg)
        extract(bp, sg)
    pltpu.sync_copy(ctile, cen_out.at[pl.ds(tbase, PER_W)])


@jax.jit
def _sc_center(cen, w_in_t, w_tail):
    f = pl.kernel(
        _sc_center_body,
        out_type=jax.ShapeDtypeStruct((B, D), jnp.float32),
        mesh=plsc.VectorSubcoreMesh(core_axis_name="c", subcore_axis_name="s"),
        compiler_params=pltpu.CompilerParams(needs_layout_passes=False),
        scratch_types=[
            pltpu.VMEM((PER_W + L,), jnp.int32),
            pltpu.VMEM((GB * D, 128), jnp.float32),
            pltpu.VMEM((PER_W, D), jnp.float32),
            pltpu.VMEM((64, 128), jnp.float32),
            [pltpu.SemaphoreType.DMA] * GB,
        ],
    )
    return f(cen, w_in_t, w_tail)


def _sc_scores_body(cen_hbm, ctx_hbm, neg_hbm, wout_hbm,
                    pos_out, neg_out,
                    xidx, nidx, crows, prows, nrows, psc, nsc,
                    isem, gsem):
    wid = lax.axis_index("s") * NC + lax.axis_index("c")
    lane15 = lax.iota(jnp.int32, L) == (L - 1)

    def stage_idx(c, p):
        base = wid * PER_W + c * CHUNK
        cps = [pltpu.async_copy(ctx_hbm.at[pl.ds(base, CHUNK)], xidx[p], isem),
               pltpu.async_copy(cen_hbm.at[pl.ds(base, CHUNK)], crows[p], isem)]
        for k in range(K):
            cps.append(pltpu.async_copy(
                neg_hbm.at[pl.ds(k * B + base, CHUNK)],
                nidx[p].at[pl.ds(k * CHUNK, CHUNK)], isem))
        return cps

    def gather_rows(p):
        cps = [pltpu.async_copy(wout_hbm.at[xidx[p]], prows[p], gsem)]
        for j in range(NGATH):
            cps.append(pltpu.async_copy(
                wout_hbm.at[nidx[p].at[pl.ds(j * 128, 128)]],
                nrows[p].at[pl.ds(j * 128, 128)], gsem))
        return cps

    def compute(c, p):
        def brow(b, _):
            cvs = [crows[p][b, pl.ds(j * L, L)] for j in range(D // L)]
            pvs = [prows[p][b, pl.ds(j * L, L)] for j in range(D // L)]
            s = plsc.cumsum(sum(cv * pv for cv, pv in zip(cvs, pvs)))
            plsc.store_scatter(psc, [jnp.full((L,), c * CHUNK + b, jnp.int32)],
                               s, mask=lane15)
            for k in range(K):
                nvs = [nrows[p][k * CHUNK + b, pl.ds(j * L, L)]
                       for j in range(D // L)]
                t = plsc.cumsum(sum(cv * nv for cv, nv in zip(cvs, nvs)))
                plsc.store_scatter(
                    nsc, [jnp.full((L,), k * PER_W + c * CHUNK + b, jnp.int32)],
                    t, mask=lane15)
            return 0

        lax.fori_loop(0, CHUNK, brow, 0)

    idx_cps = stage_idx(0, 0)
    gat_cps = None
    for c in range(NSTEP):
        p = c % 2
        for cp in idx_cps:
            cp.wait()
        if gat_cps:
            for cp in gat_cps:
                cp.wait()
        idx_cps = stage_idx(c + 1, 1 - p) if c + 1 < NSTEP else []
        gat_cps = gather_rows(p)
        if c > 0:
            compute(c - 1, 1 - p)
    for cp in gat_cps:
        cp.wait()
    compute(NSTEP - 1, (NSTEP - 1) % 2)

    pltpu.sync_copy(psc, pos_out.at[pl.ds(wid * PER_W, PER_W)])
    for k in range(K):
        pltpu.sync_copy(nsc.at[pl.ds(k * PER_W, PER_W)],
                        neg_out.at[pl.ds(k * B + wid * PER_W, PER_W)])


@jax.jit
def _sc_scores(cen_emb, ctx, neg1d, w_out):
    f = pl.kernel(
        _sc_scores_body,
        out_type=(jax.ShapeDtypeStruct((B,), jnp.float32),
                  jax.ShapeDtypeStruct((B * K,), jnp.float32)),
        mesh=plsc.VectorSubcoreMesh(core_axis_name="c", subcore_axis_name="s"),
        compiler_params=pltpu.CompilerParams(needs_layout_passes=False,
                                             use_tc_tiling_on_sc=False),
        scratch_types=[
            [pltpu.VMEM((CHUNK,), jnp.int32)] * 2,
            [pltpu.VMEM((NNEG,), jnp.int32)] * 2,
            [pltpu.VMEM((CHUNK, D), jnp.float32)] * 2,
            [pltpu.VMEM((CHUNK, D), jnp.float32)] * 2,
            [pltpu.VMEM((NNEG, D), jnp.float32)] * 2,
            pltpu.VMEM((PER_W,), jnp.float32),
            pltpu.VMEM((K * PER_W,), jnp.float32),
            pltpu.SemaphoreType.DMA,
            pltpu.SemaphoreType.DMA,
        ],
    )
    return f(cen_emb, ctx, neg1d, w_out)


def _tc_loss_body(pos_ref, neg_ref, out_ref):
    pls = jax.nn.log_sigmoid(pos_ref[...])
    nls = jax.nn.log_sigmoid(-neg_ref[...])
    out_ref[0, 0] = -(jnp.sum(pls) + jnp.sum(nls)) / B


def _tc_loss(pos2d, neg2d):
    return pl.pallas_call(
        _tc_loss_body,
        out_shape=jax.ShapeDtypeStruct((1, 1), jnp.float32),
        out_specs=pl.BlockSpec(memory_space=pltpu.SMEM),
    )(pos2d, neg2d)


def kernel(center, context, negatives, W_in, W_out):
    cen = center.astype(jnp.int32)
    ctx = context.astype(jnp.int32)
    neg = negatives.astype(jnp.int32).T.reshape(B * K)
    w_tail = jnp.pad(W_in[VOCAB - 64:], ((0, 0), (0, 128 - D)))
    cen_emb = _sc_center(cen, W_in.T, w_tail)
    pos_s, neg_s = _sc_scores(cen_emb, ctx, neg, W_out)
    loss = _tc_loss(pos_s.reshape(B // 128, 128),
                    neg_s.reshape(B * K // 128, 128))
    return loss[0, 0]


# speedup vs baseline: 1.9873x; 1.2741x over previous
import jax
import jax.numpy as jnp
from jax import lax
from jax.experimental import pallas as pl
from jax.experimental.pallas import tpu as pltpu
from jax.experimental.pallas import tpu_sc as plsc

VOCAB = 1000000
D = 64
B = 16384
K = 20

NC, NS, L = 2, 16, 16
NW = NC * NS
PER_W = B // NW
CHUNK = 32
NSTEP = PER_W // CHUNK
NNEG = CHUNK * K
NV = D // L

GB = 4
CLAST = (VOCAB - 128) // 128 * 128
TAIL0 = CLAST + 128
TBLK = 2048
HALF = 248 * TBLK


def _tc_pair_body(lo_ref, hi_ref, out_ref):
    out_ref[:, 0:D] = lo_ref[...].T
    out_ref[:, D:2 * D] = hi_ref[...].T


def _tc_pair(w_t):
    grid = HALF // TBLK
    return pl.pallas_call(
        _tc_pair_body,
        grid=(grid,),
        in_specs=[pl.BlockSpec((D, TBLK), lambda i: (0, i)),
                  pl.BlockSpec(
                      (D, TBLK),
                      lambda i: (0, jnp.minimum(i + HALF // TBLK,
                                                VOCAB // TBLK)))],
        out_specs=pl.BlockSpec((TBLK, 2 * D), lambda i: (i, 0)),
        out_shape=jax.ShapeDtypeStruct((HALF, 2 * D), jnp.float32),
    )(w_t, w_t)


def _sc_center_body(cen_hbm, wt_hbm, tail_hbm, cen_out,
                    cidx, blk, ctile, wtail, sems):
    wid = lax.axis_index("s") * NC + lax.axis_index("c")
    tbase = wid * PER_W
    pltpu.sync_copy(cen_hbm.at[pl.ds(tbase, PER_W)], cidx.at[pl.ds(0, PER_W)])
    pltpu.sync_copy(tail_hbm, wtail)
    lanes = lax.iota(jnp.int32, L)

    def rof(b):
        vec = cidx[pl.ds(b & ~(L - 1), L)]
        return jnp.max(jnp.where(lanes == (b & (L - 1)), vec, 0))

    def c0of(r):
        return jnp.minimum(lax.shift_right_logical(r, 7), CLAST // 128) * 128

    def tiles(b, sg):
        c0 = pl.multiple_of(c0of(rof(b)), 128)
        return [(wt_hbm.at[pl.ds(t * 8, 8), pl.ds(c0, 128)],
                 blk.at[pl.ds(sg * D + t * 8, 8)]) for t in range(D // 8)]

    def issue(b, sg):
        for src, dst in tiles(b, sg):
            pltpu.async_copy(src, dst, sems[sg])

    def drain(b, sg):
        for src, dst in tiles(b, sg):
            pltpu.make_async_copy(src, dst, sems[sg]).wait()

    def extract(b, sg):
        r = rof(b)
        col = jnp.minimum(r - c0of(r), 127)
        rt = jnp.minimum(jnp.maximum(r - TAIL0, 0), 63)
        is_tail = r >= TAIL0
        csp = jnp.full((L,), col, jnp.int32)
        for j in range(NV):
            v = plsc.load_gather(blk, [sg * D + j * L + lanes, csp])
            tv = wtail[rt, pl.ds(j * L, L)]
            ctile[b, pl.ds(j * L, L)] = jnp.where(is_tail, tv, v)

    for sg in range(GB):
        issue(sg, sg)

    def step(g, _):
        for sg in range(GB):
            bp = (g - 1) * GB + sg
            drain(bp, sg)
            extract(bp, sg)
            issue(g * GB + sg, sg)
        return 0

    lax.fori_loop(1, PER_W // GB, step, 0)
    for sg in range(GB):
        bp = PER_W - GB + sg
        drain(bp, sg)
        extract(bp, sg)
    pltpu.sync_copy(ctile, cen_out.at[pl.ds(tbase, PER_W)])


@jax.jit
def _sc_center(cen, w_in_t, w_tail):
    f = pl.kernel(
        _sc_center_body,
        out_type=jax.ShapeDtypeStruct((B, D), jnp.float32),
        mesh=plsc.VectorSubcoreMesh(core_axis_name="c", subcore_axis_name="s"),
        compiler_params=pltpu.CompilerParams(needs_layout_passes=False),
        scratch_types=[
            pltpu.VMEM((PER_W + L,), jnp.int32),
            pltpu.VMEM((GB * D, 128), jnp.float32),
            pltpu.VMEM((PER_W, D), jnp.float32),
            pltpu.VMEM((64, 128), jnp.float32),
            [pltpu.SemaphoreType.DMA] * GB,
        ],
    )
    return f(cen, w_in_t, w_tail)


def _sc_scores_body(cen_hbm, ctx_hbm, neg_hbm, wp_hbm,
                    pos_out, neg_out,
                    xidx, nidx, xp_i, np_i, crows, prows, nrows, psc, nsc,
                    isem, gsem):
    wid = lax.axis_index("s") * NC + lax.axis_index("c")
    lane15 = lax.iota(jnp.int32, L) == (L - 1)

    def stage_idx(c, p):
        base = wid * PER_W + c * CHUNK
        cps = [pltpu.async_copy(ctx_hbm.at[pl.ds(base, CHUNK)], xidx[p], isem),
               pltpu.async_copy(cen_hbm.at[pl.ds(base, CHUNK)], crows[p], isem)]
        for k in range(K):
            cps.append(pltpu.async_copy(
                neg_hbm.at[pl.ds(k * B + base, CHUNK)],
                nidx[p].at[pl.ds(k * CHUNK, CHUNK)], isem))
        return cps

    def pmod(v):
        return jnp.where(v >= HALF, v - HALF, v)

    def gather_rows(p):
        for j in range(CHUNK // L):
            xp_i[pl.ds(j * L, L)] = pmod(xidx[p][pl.ds(j * L, L)])
        for j in range(NNEG // L):
            np_i[pl.ds(j * L, L)] = pmod(nidx[p][pl.ds(j * L, L)])
        cps = [pltpu.async_copy(wp_hbm.at[xp_i], prows, gsem)]
        for j in range(NNEG // 128):
            cps.append(pltpu.async_copy(
                wp_hbm.at[np_i.at[pl.ds(j * 128, 128)]],
                nrows.at[pl.ds(j * 128, 128)], gsem))
        for cp in cps:
            cp.wait()

    def compute(c, p):
        def brow(b, _):
            cvs = [crows[p][b, pl.ds(j * L, L)] for j in range(NV)]
            hx = (plsc.load_gather(xidx[p], [jnp.full((L,), b, jnp.int32)])
                  >= HALF).astype(jnp.float32)
            pv = [prows[b, pl.ds(j * L, L)] for j in range(2 * NV)]
            slo = sum(cv * v for cv, v in zip(cvs, pv[:NV]))
            shi = sum(cv * v for cv, v in zip(cvs, pv[NV:]))
            s = plsc.cumsum(slo) * (1.0 - hx) + plsc.cumsum(shi) * hx
            plsc.store_scatter(psc, [jnp.full((L,), c * CHUNK + b, jnp.int32)],
                               s, mask=lane15)

            def kstep(k, _):
                hn = (plsc.load_gather(
                    nidx[p], [jnp.full((L,), k * CHUNK + b, jnp.int32)])
                    >= HALF).astype(jnp.float32)
                nv = [nrows[k * CHUNK + b, pl.ds(j * L, L)]
                      for j in range(2 * NV)]
                tlo = sum(cv * v for cv, v in zip(cvs, nv[:NV]))
                thi = sum(cv * v for cv, v in zip(cvs, nv[NV:]))
                t = plsc.cumsum(tlo) * (1.0 - hn) + plsc.cumsum(thi) * hn
                plsc.store_scatter(
                    nsc, [jnp.full((L,), k * PER_W + c * CHUNK + b, jnp.int32)],
                    t, mask=lane15)
                return 0

            lax.fori_loop(0, K, kstep, 0)
            return 0

        lax.fori_loop(0, CHUNK, brow, 0)

    idx_cps = stage_idx(0, 0)
    for c in range(NSTEP):
        p = c % 2
        for cp in idx_cps:
            cp.wait()
        idx_cps = stage_idx(c + 1, 1 - p) if c + 1 < NSTEP else []
        gather_rows(p)
        compute(c, p)

    pltpu.sync_copy(psc, pos_out.at[pl.ds(wid * PER_W, PER_W)])
    for k in range(K):
        pltpu.sync_copy(nsc.at[pl.ds(k * PER_W, PER_W)],
                        neg_out.at[pl.ds(k * B + wid * PER_W, PER_W)])


@jax.jit
def _sc_scores(cen_emb, ctx, neg1d, w_pair):
    f = pl.kernel(
        _sc_scores_body,
        out_type=(jax.ShapeDtypeStruct((B,), jnp.float32),
                  jax.ShapeDtypeStruct((B * K,), jnp.float32)),
        mesh=plsc.VectorSubcoreMesh(core_axis_name="c", subcore_axis_name="s"),
        compiler_params=pltpu.CompilerParams(needs_layout_passes=False),
        scratch_types=[
            [pltpu.VMEM((CHUNK,), jnp.int32)] * 2,
            [pltpu.VMEM((NNEG,), jnp.int32)] * 2,
            pltpu.VMEM((CHUNK,), jnp.int32),
            pltpu.VMEM((NNEG,), jnp.int32),
            [pltpu.VMEM((CHUNK, D), jnp.float32)] * 2,
            pltpu.VMEM((CHUNK, 2 * D), jnp.float32),
            pltpu.VMEM((NNEG, 2 * D), jnp.float32),
            pltpu.VMEM((PER_W,), jnp.float32),
            pltpu.VMEM((K * PER_W,), jnp.float32),
            pltpu.SemaphoreType.DMA,
            pltpu.SemaphoreType.DMA,
        ],
    )
    return f(cen_emb, ctx, neg1d, w_pair)


def _tc_loss_body(pos_ref, neg_ref, out_ref):
    pls = jax.nn.log_sigmoid(pos_ref[...])
    nls = jax.nn.log_sigmoid(-neg_ref[...])
    out_ref[0, 0] = -(jnp.sum(pls) + jnp.sum(nls)) / B


def _tc_loss(pos2d, neg2d):
    return pl.pallas_call(
        _tc_loss_body,
        out_shape=jax.ShapeDtypeStruct((1, 1), jnp.float32),
        out_specs=pl.BlockSpec(memory_space=pltpu.SMEM),
    )(pos2d, neg2d)


def kernel(center, context, negatives, W_in, W_out):
    cen = center.astype(jnp.int32)
    ctx = context.astype(jnp.int32)
    neg = negatives.astype(jnp.int32).T.reshape(B * K)
    w_tail = jnp.pad(W_in[VOCAB - 64:], ((0, 0), (0, 128 - D)))
    cen_emb = _sc_center(cen, W_in.T, w_tail)
    w_pair = _tc_pair(W_out.T)
    pos_s, neg_s = _sc_scores(cen_emb, ctx, neg, w_pair)
    loss = _tc_loss(pos_s.reshape(B // 128, 128),
                    neg_s.reshape(B * K // 128, 128))
    return loss[0, 0]


# speedup vs baseline: 2.1940x; 1.1040x over previous
import jax
import jax.numpy as jnp
from jax import lax
from jax.experimental import pallas as pl
from jax.experimental.pallas import tpu as pltpu
from jax.experimental.pallas import tpu_sc as plsc

VOCAB = 1000000
D = 64
B = 16384
K = 20

NC, NS, L = 2, 16, 16
NW = NC * NS
PER_W = B // NW
CHUNK = 16
NSTEP = PER_W // CHUNK
NROW = CHUNK * (K + 1)
NSLC = [(o, min(128, NROW - o)) for o in range(0, NROW, 128)]
NV = D // L

GB = 4
CLAST = (VOCAB - 128) // 128 * 128
TAIL0 = CLAST + 128
TBLK = 2048
HALF = 248 * TBLK


def _tc_pair_body(lo_ref, hi_ref, out_ref):
    out_ref[:, 0:D] = lo_ref[...].T
    out_ref[:, D:2 * D] = hi_ref[...].T


def _tc_pair(w_t):
    grid = HALF // TBLK
    return pl.pallas_call(
        _tc_pair_body,
        grid=(grid,),
        in_specs=[pl.BlockSpec((D, TBLK), lambda i: (0, i)),
                  pl.BlockSpec(
                      (D, TBLK),
                      lambda i: (0, jnp.minimum(i + HALF // TBLK,
                                                VOCAB // TBLK)))],
        out_specs=pl.BlockSpec((TBLK, 2 * D), lambda i: (i, 0)),
        out_shape=jax.ShapeDtypeStruct((HALF, 2 * D), jnp.float32),
    )(w_t, w_t)


def _sc_center_body(cen_hbm, wt_hbm, tail_hbm, cen_out,
                    cidx, blk, ctile, wtail, sems):
    wid = lax.axis_index("s") * NC + lax.axis_index("c")
    tbase = wid * PER_W
    pltpu.sync_copy(cen_hbm.at[pl.ds(tbase, PER_W)], cidx.at[pl.ds(0, PER_W)])
    pltpu.sync_copy(tail_hbm, wtail)
    lanes = lax.iota(jnp.int32, L)

    def rof(b):
        vec = cidx[pl.ds(b & ~(L - 1), L)]
        return jnp.max(jnp.where(lanes == (b & (L - 1)), vec, 0))

    def c0of(r):
        return jnp.minimum(lax.shift_right_logical(r, 7), CLAST // 128) * 128

    def tiles(b, sg):
        c0 = pl.multiple_of(c0of(rof(b)), 128)
        return [(wt_hbm.at[pl.ds(t * 8, 8), pl.ds(c0, 128)],
                 blk.at[pl.ds(sg * D + t * 8, 8)]) for t in range(D // 8)]

    def issue(b, sg):
        for src, dst in tiles(b, sg):
            pltpu.async_copy(src, dst, sems[sg])

    def drain(b, sg):
        for src, dst in tiles(b, sg):
            pltpu.make_async_copy(src, dst, sems[sg]).wait()

    def extract(b, sg):
        r = rof(b)
        col = jnp.minimum(r - c0of(r), 127)
        rt = jnp.minimum(jnp.maximum(r - TAIL0, 0), 63)
        is_tail = r >= TAIL0
        csp = jnp.full((L,), col, jnp.int32)
        for j in range(NV):
            v = plsc.load_gather(blk, [sg * D + j * L + lanes, csp])
            tv = wtail[rt, pl.ds(j * L, L)]
            ctile[b, pl.ds(j * L, L)] = jnp.where(is_tail, tv, v)

    for sg in range(GB):
        issue(sg, sg)

    def step(g, _):
        for sg in range(GB):
            bp = (g - 1) * GB + sg
            drain(bp, sg)
            extract(bp, sg)
            issue(g * GB + sg, sg)
        return 0

    lax.fori_loop(1, PER_W // GB, step, 0)
    for sg in range(GB):
        bp = PER_W - GB + sg
        drain(bp, sg)
        extract(bp, sg)
    pltpu.sync_copy(ctile, cen_out.at[pl.ds(tbase, PER_W)])


@jax.jit
def _sc_center(cen, w_in_t, w_tail):
    f = pl.kernel(
        _sc_center_body,
        out_type=jax.ShapeDtypeStruct((B, D), jnp.float32),
        mesh=plsc.VectorSubcoreMesh(core_axis_name="c", subcore_axis_name="s"),
        compiler_params=pltpu.CompilerParams(needs_layout_passes=False),
        scratch_types=[
            pltpu.VMEM((PER_W + L,), jnp.int32),
            pltpu.VMEM((GB * D, 128), jnp.float32),
            pltpu.VMEM((PER_W, D), jnp.float32),
            pltpu.VMEM((64, 128), jnp.float32),
            [pltpu.SemaphoreType.DMA] * GB,
        ],
    )
    return f(cen, w_in_t, w_tail)


def _sc_scores_body(cen_hbm, ctx_hbm, neg_hbm, wp_hbm,
                    pos_out, neg_out,
                    nidx, np_i, crows, nrows, allsc,
                    isem, gsem):
    wid = lax.axis_index("s") * NC + lax.axis_index("c")
    lane15 = lax.iota(jnp.int32, L) == (L - 1)

    def stage_idx(c, p):
        base = wid * PER_W + c * CHUNK
        cps = [pltpu.async_copy(ctx_hbm.at[pl.ds(base, CHUNK)],
                                nidx[p].at[pl.ds(K * CHUNK, CHUNK)], isem),
               pltpu.async_copy(cen_hbm.at[pl.ds(base, CHUNK)], crows[p], isem)]

        def kcopy(k, _):
            pltpu.async_copy(neg_hbm.at[pl.ds(k * B + base, CHUNK)],
                             nidx[p].at[pl.ds(k * CHUNK, CHUNK)], isem)
            return 0

        lax.fori_loop(0, K, kcopy, 0)
        cps.append(pltpu.make_async_copy(
            neg_hbm.at[pl.ds(base * K, K * CHUNK)],
            nidx[p].at[pl.ds(0, K * CHUNK)], isem))
        return cps

    def gather_rows(p):
        def shift(j, _):
            v = nidx[p][pl.ds(j * L, L)]
            np_i[p][pl.ds(j * L, L)] = jnp.where(v >= HALF, v - HALF, v)
            return 0

        lax.fori_loop(0, NROW // L, shift, 0)
        cps = []
        for o, n in NSLC:
            cps.append(pltpu.async_copy(
                wp_hbm.at[np_i[p].at[pl.ds(o, n)]],
                nrows[p].at[pl.ds(o, n)], gsem))
        return cps

    def compute(c, p):
        def brow(b, _):
            cvs = [crows[p][b, pl.ds(j * L, L)] for j in range(NV)]

            def kstep(k, _):
                hn = (plsc.load_gather(
                    nidx[p], [jnp.full((L,), k * CHUNK + b, jnp.int32)])
                    >= HALF).astype(jnp.float32)
                nv = [nrows[p][k * CHUNK + b, pl.ds(j * L, L)]
                      for j in range(2 * NV)]
                tlo = sum(cv * v for cv, v in zip(cvs, nv[:NV]))
                thi = sum(cv * v for cv, v in zip(cvs, nv[NV:]))
                t = plsc.cumsum(tlo) * (1.0 - hn) + plsc.cumsum(thi) * hn
                plsc.store_scatter(
                    allsc, [jnp.full((L,), k * PER_W + c * CHUNK + b,
                                     jnp.int32)],
                    t, mask=lane15)
                return 0

            lax.fori_loop(0, K + 1, kstep, 0)
            return 0

        lax.fori_loop(0, CHUNK, brow, 0)

    idx_cps = stage_idx(0, 0)
    gat_cps = None
    for c in range(NSTEP):
        p = c % 2
        for cp in idx_cps:
            cp.wait()
        if gat_cps:
            for cp in gat_cps:
                cp.wait()
        idx_cps = stage_idx(c + 1, 1 - p) if c + 1 < NSTEP else []
        gat_cps = gather_rows(p)
        if c > 0:
            compute(c - 1, 1 - p)
    for cp in gat_cps:
        cp.wait()
    compute(NSTEP - 1, (NSTEP - 1) % 2)

    pltpu.sync_copy(allsc.at[pl.ds(K * PER_W, PER_W)],
                    pos_out.at[pl.ds(wid * PER_W, PER_W)])
    for k in range(K):
        pltpu.sync_copy(allsc.at[pl.ds(k * PER_W, PER_W)],
                        neg_out.at[pl.ds(k * B + wid * PER_W, PER_W)])


@jax.jit
def _sc_scores(cen_emb, ctx, neg1d, w_pair):
    f = pl.kernel(
        _sc_scores_body,
        out_type=(jax.ShapeDtypeStruct((B,), jnp.float32),
                  jax.ShapeDtypeStruct((B * K,), jnp.float32)),
        mesh=plsc.VectorSubcoreMesh(core_axis_name="c", subcore_axis_name="s"),
        compiler_params=pltpu.CompilerParams(needs_layout_passes=False),
        scratch_types=[
            [pltpu.VMEM((NROW,), jnp.int32)] * 2,
            [pltpu.VMEM((NROW,), jnp.int32)] * 2,
            [pltpu.VMEM((CHUNK, D), jnp.float32)] * 2,
            [pltpu.VMEM((NROW, 2 * D), jnp.float32)] * 2,
            pltpu.VMEM(((K + 1) * PER_W,), jnp.float32),
            pltpu.SemaphoreType.DMA,
            pltpu.SemaphoreType.DMA,
        ],
    )
    return f(cen_emb, ctx, neg1d, w_pair)


def _tc_loss_body(pos_ref, neg_ref, out_ref):
    pls = jax.nn.log_sigmoid(pos_ref[...])
    nls = jax.nn.log_sigmoid(-neg_ref[...])
    out_ref[0, 0] = -(jnp.sum(pls) + jnp.sum(nls)) / B


def _tc_loss(pos2d, neg2d):
    return pl.pallas_call(
        _tc_loss_body,
        out_shape=jax.ShapeDtypeStruct((1, 1), jnp.float32),
        out_specs=pl.BlockSpec(memory_space=pltpu.SMEM),
    )(pos2d, neg2d)


def kernel(center, context, negatives, W_in, W_out):
    cen = center.astype(jnp.int32)
    ctx = context.astype(jnp.int32)
    neg = negatives.astype(jnp.int32).T.reshape(B * K)
    w_tail = jnp.pad(W_in[VOCAB - 64:], ((0, 0), (0, 128 - D)))
    cen_emb = _sc_center(cen, W_in.T, w_tail)
    w_pair = _tc_pair(W_out.T)
    pos_s, neg_s = _sc_scores(cen_emb, ctx, neg, w_pair)
    loss = _tc_loss(pos_s.reshape(B // 128, 128),
                    neg_s.reshape(B * K // 128, 128))
    return loss[0, 0]
